# Initial kernel scaffold; baseline (speedup 1.0000x reference)
import jax
import jax.numpy as jnp
from jax.experimental import pallas as pl


def kernel(card_indices, stages, visibility, order, rank_emb, suit_emb, stage_emb, visibility_emb, order_emb):
    raise NotImplementedError("write your pallas kernel here")


# speedup vs baseline: 12.6214x; 12.6214x over previous
import functools

import jax
import jax.numpy as jnp
from jax import lax
from jax.experimental import pallas as pl
from jax.experimental.pallas import tpu as pltpu
from jax.experimental.pallas import tpu_sc as plsc

D = 128
N_T = 3120
OFF_RANK, OFF_SUIT, OFF_STAGE, OFF_VIS, OFF_ORDER = 0, 16, 24, 32, 40
CAT_ROWS = 48


def _prep_body(card_ref, st_ref, vi_ref, od_ref, cat_ref, cidx_ref, t_ref):
    i = pl.program_id(0)
    cidx_ref[...] = (
        card_ref[...] * 60 + st_ref[...] * 15 + vi_ref[...] * 5 + od_ref[...]
    )
    @pl.when(i == 0)
    def _():
        row = lax.broadcasted_iota(jnp.int32, (N_T, 1), 0)
        card = row // 60
        svo = row - card * 60
        stage = svo // 15
        rem = svo - stage * 15
        vis = rem // 5
        order = rem - vis * 5
        cols = lax.broadcasted_iota(jnp.int32, (N_T, CAT_ROWS), 1)
        onehot = (
            (cols == OFF_RANK + card % 13).astype(jnp.float32)
            + (cols == OFF_SUIT + card // 13).astype(jnp.float32)
            + (cols == OFF_STAGE + stage).astype(jnp.float32)
            + (cols == OFF_VIS + vis).astype(jnp.float32)
            + (cols == OFF_ORDER + order).astype(jnp.float32)
        )
        t_ref[...] = jnp.dot(onehot, cat_ref[...],
                             preferred_element_type=jnp.float32)


def _prep(card, st, vi, od, cat):
    rows = card.shape[0]
    grid = 16
    blk = rows // grid
    return pl.pallas_call(
        _prep_body,
        grid=(grid,),
        in_specs=[
            pl.BlockSpec((blk, D), lambda i: (i, 0)),
            pl.BlockSpec((blk, D), lambda i: (i, 0)),
            pl.BlockSpec((blk, D), lambda i: (i, 0)),
            pl.BlockSpec((blk, D), lambda i: (i, 0)),
            pl.BlockSpec((CAT_ROWS, D), lambda i: (0, 0)),
        ],
        out_specs=[
            pl.BlockSpec((blk, D), lambda i: (i, 0)),
            pl.BlockSpec((N_T, D), lambda i: (0, 0)),
        ],
        out_shape=[
            jax.ShapeDtypeStruct((rows, D), jnp.int32),
            jax.ShapeDtypeStruct((N_T, D), jnp.float32),
        ],
    )(card, st, vi, od, cat)


def _make_sc_gather(n_rows):
    info = plsc.get_sparse_core_info()
    nc, ns = info.num_cores, info.num_subcores
    nw = nc * ns
    rw = n_rows // nw
    blk = 128
    nblk = rw // blk
    mesh = plsc.VectorSubcoreMesh(core_axis_name="c", subcore_axis_name="s")

    @functools.partial(
        pl.kernel,
        out_type=jax.ShapeDtypeStruct((n_rows, D), jnp.float32),
        mesh=mesh,
        scratch_types=[
            pltpu.VMEM((nblk, blk), jnp.int32),
            pltpu.VMEM((blk, D), jnp.float32),
            pltpu.SemaphoreType.DMA,
        ],
    )
    def sc_gather(t_hbm, idx_hbm, out_hbm, idx_v, rows_v, gsem):
        wid = lax.axis_index("s") * nc + lax.axis_index("c")
        base = wid * rw
        pltpu.sync_copy(idx_hbm.at[wid], idx_v)

        def body(j, carry):
            pltpu.async_copy(t_hbm.at[idx_v.at[j]], rows_v, gsem).wait()
            pltpu.sync_copy(rows_v, out_hbm.at[pl.ds(base + j * blk, blk)])
            return carry

        lax.fori_loop(0, nblk, body, 0)

    return sc_gather, nw, nblk, blk


def kernel(card_indices, stages, visibility, order,
           rank_emb, suit_emb, stage_emb, visibility_emb, order_emb):
    B, L = card_indices.shape
    n = B * L
    rows = n // D

    card = card_indices.astype(jnp.int32).reshape(rows, D)
    st = stages.astype(jnp.int32).reshape(rows, D)
    vi = visibility.astype(jnp.int32).reshape(rows, D)
    od = order.astype(jnp.int32).reshape(rows, D)

    def pad8(t, r):
        return jnp.pad(t, ((0, r - t.shape[0]), (0, 0)))

    cat = jnp.concatenate([
        pad8(rank_emb, 16), pad8(suit_emb, 8), pad8(stage_emb, 8),
        pad8(visibility_emb, 8), pad8(order_emb, 8),
    ], axis=0)

    cidx, table = _prep(card, st, vi, od, cat)

    sc_gather, nw, nblk, blk = _make_sc_gather(n)
    idx3 = cidx.reshape(nw, nblk, blk)
    out = sc_gather(table, idx3)
    return out.reshape(B, L, D)


# speedup vs baseline: 14.0348x; 1.1120x over previous
import functools

import jax
import jax.numpy as jnp
from jax import lax
from jax.experimental import pallas as pl
from jax.experimental.pallas import tpu as pltpu
from jax.experimental.pallas import tpu_sc as plsc

D = 128
N_T = 3120
OFF_RANK, OFF_SUIT, OFF_STAGE, OFF_VIS, OFF_ORDER = 0, 16, 24, 32, 40
CAT_ROWS = 48


def _prep_body(card_ref, st_ref, vi_ref, od_ref, cat_ref, cidx_ref, t_ref):
    i = pl.program_id(0)
    cidx_ref[...] = (
        card_ref[...] * 60 + st_ref[...] * 15 + vi_ref[...] * 5 + od_ref[...]
    )
    @pl.when(i == 0)
    def _():
        row = lax.broadcasted_iota(jnp.int32, (N_T, 1), 0)
        card = row // 60
        svo = row - card * 60
        stage = svo // 15
        rem = svo - stage * 15
        vis = rem // 5
        order = rem - vis * 5
        cols = lax.broadcasted_iota(jnp.int32, (N_T, CAT_ROWS), 1)
        onehot = (
            (cols == OFF_RANK + card % 13).astype(jnp.float32)
            + (cols == OFF_SUIT + card // 13).astype(jnp.float32)
            + (cols == OFF_STAGE + stage).astype(jnp.float32)
            + (cols == OFF_VIS + vis).astype(jnp.float32)
            + (cols == OFF_ORDER + order).astype(jnp.float32)
        )
        t_ref[...] = jnp.dot(onehot, cat_ref[...],
                             preferred_element_type=jnp.float32)


def _prep(card, st, vi, od, cat):
    rows = card.shape[0]
    grid = 16
    blk = rows // grid
    return pl.pallas_call(
        _prep_body,
        grid=(grid,),
        in_specs=[
            pl.BlockSpec((blk, D), lambda i: (i, 0)),
            pl.BlockSpec((blk, D), lambda i: (i, 0)),
            pl.BlockSpec((blk, D), lambda i: (i, 0)),
            pl.BlockSpec((blk, D), lambda i: (i, 0)),
            pl.BlockSpec((CAT_ROWS, D), lambda i: (0, 0)),
        ],
        out_specs=[
            pl.BlockSpec((blk, D), lambda i: (i, 0)),
            pl.BlockSpec((N_T, D), lambda i: (0, 0)),
        ],
        out_shape=[
            jax.ShapeDtypeStruct((rows, D), jnp.int32),
            jax.ShapeDtypeStruct((N_T, D), jnp.float32),
        ],
    )(card, st, vi, od, cat)


def _make_sc_gather(n_rows):
    info = plsc.get_sparse_core_info()
    nc, ns = info.num_cores, info.num_subcores
    nw = nc * ns
    rw = n_rows // nw
    blk = 128
    nblk = rw // blk
    nbuf = 5
    lookahead = 3
    ngrp = nblk // nbuf
    mesh = plsc.VectorSubcoreMesh(core_axis_name="c", subcore_axis_name="s")

    @functools.partial(
        pl.kernel,
        out_type=jax.ShapeDtypeStruct((n_rows, D), jnp.float32),
        mesh=mesh,
        scratch_types=[
            pltpu.VMEM((nblk, blk), jnp.int32),
            pltpu.VMEM((nbuf, blk, D), jnp.float32),
            pltpu.SemaphoreType.DMA,
            [pltpu.SemaphoreType.DMA] * nbuf,
        ],
    )
    def sc_gather(t_hbm, idx_hbm, out_hbm, idx_v, rows_v, gsem, wsems):
        wid = lax.axis_index("s") * nc + lax.axis_index("c")
        base = wid * rw
        pltpu.sync_copy(idx_hbm.at[wid], idx_v)

        for m in range(lookahead):
            pltpu.async_copy(t_hbm.at[idx_v.at[m]], rows_v.at[m % nbuf], gsem)

        def group(g, carry):
            for b in range(nbuf):
                j = g * nbuf + b
                m_b = (b + lookahead) % nbuf
                mj = j + lookahead

                @pl.when(mj < nblk)
                def _():
                    @pl.when(mj >= nbuf)
                    def _():
                        pltpu.make_async_copy(
                            rows_v.at[m_b],
                            out_hbm.at[pl.ds(base, blk)],
                            wsems[m_b],
                        ).wait()
                    pltpu.async_copy(t_hbm.at[idx_v.at[mj]],
                                     rows_v.at[m_b], gsem)

                pltpu.make_async_copy(
                    t_hbm.at[idx_v.at[j]], rows_v.at[b], gsem
                ).wait()
                pltpu.async_copy(rows_v.at[b],
                                 out_hbm.at[pl.ds(base + j * blk, blk)],
                                 wsems[b])
            return carry

        lax.fori_loop(0, ngrp, group, 0)

        for b in range(nbuf):
            pltpu.make_async_copy(
                rows_v.at[b], out_hbm.at[pl.ds(base, blk)], wsems[b]
            ).wait()

    return sc_gather, nw, nblk, blk


def kernel(card_indices, stages, visibility, order,
           rank_emb, suit_emb, stage_emb, visibility_emb, order_emb):
    B, L = card_indices.shape
    n = B * L
    rows = n // D

    card = card_indices.astype(jnp.int32).reshape(rows, D)
    st = stages.astype(jnp.int32).reshape(rows, D)
    vi = visibility.astype(jnp.int32).reshape(rows, D)
    od = order.astype(jnp.int32).reshape(rows, D)

    def pad8(t, r):
        return jnp.pad(t, ((0, r - t.shape[0]), (0, 0)))

    cat = jnp.concatenate([
        pad8(rank_emb, 16), pad8(suit_emb, 8), pad8(stage_emb, 8),
        pad8(visibility_emb, 8), pad8(order_emb, 8),
    ], axis=0)

    cidx, table = _prep(card, st, vi, od, cat)

    sc_gather, nw, nblk, blk = _make_sc_gather(n)
    idx3 = cidx.reshape(nw, nblk, blk)
    out = sc_gather(table, idx3)
    return out.reshape(B, L, D)


# speedup vs baseline: 38.7297x; 2.7595x over previous
import functools

import jax
import jax.numpy as jnp
from jax import lax
from jax.experimental import pallas as pl
from jax.experimental.pallas import tpu as pltpu
from jax.experimental.pallas import tpu_sc as plsc

D = 128
N_T = 3120
OFF_RANK, OFF_SUIT, OFF_STAGE, OFF_VIS, OFF_ORDER = 0, 16, 24, 32, 40
CAT_ROWS = 48


def _prep_body(card_ref, st_ref, vi_ref, od_ref, cat_ref, cidx_ref, t_ref):
    i = pl.program_id(0)
    cidx_ref[...] = (
        card_ref[...] * 60 + st_ref[...] * 15 + vi_ref[...] * 5 + od_ref[...]
    )
    @pl.when(i == 0)
    def _():
        row = lax.broadcasted_iota(jnp.int32, (N_T, 1), 0)
        card = row // 60
        svo = row - card * 60
        stage = svo // 15
        rem = svo - stage * 15
        vis = rem // 5
        order = rem - vis * 5
        cols = lax.broadcasted_iota(jnp.int32, (N_T, CAT_ROWS), 1)
        onehot = (
            (cols == OFF_RANK + card % 13).astype(jnp.float32)
            + (cols == OFF_SUIT + card // 13).astype(jnp.float32)
            + (cols == OFF_STAGE + stage).astype(jnp.float32)
            + (cols == OFF_VIS + vis).astype(jnp.float32)
            + (cols == OFF_ORDER + order).astype(jnp.float32)
        )
        t_ref[...] = jnp.dot(onehot, cat_ref[...],
                             preferred_element_type=jnp.float32)


def _prep(card, st, vi, od, cat):
    rows = card.shape[0]
    grid = 16
    blk = rows // grid
    return pl.pallas_call(
        _prep_body,
        grid=(grid,),
        in_specs=[
            pl.BlockSpec((blk, D), lambda i: (i, 0)),
            pl.BlockSpec((blk, D), lambda i: (i, 0)),
            pl.BlockSpec((blk, D), lambda i: (i, 0)),
            pl.BlockSpec((blk, D), lambda i: (i, 0)),
            pl.BlockSpec((CAT_ROWS, D), lambda i: (0, 0)),
        ],
        out_specs=[
            pl.BlockSpec((blk, D), lambda i: (i, 0)),
            pl.BlockSpec((N_T, D), lambda i: (0, 0)),
        ],
        out_shape=[
            jax.ShapeDtypeStruct((rows, D), jnp.int32),
            jax.ShapeDtypeStruct((N_T, D), jnp.float32),
        ],
    )(card, st, vi, od, cat)


def _make_sc_gather(n_rows):
    info = plsc.get_sparse_core_info()
    nc, ns = info.num_cores, info.num_subcores
    nw = nc * ns
    rw = n_rows // nw
    blk = 128
    nblk = rw // blk
    nbuf = 5
    lookahead = 3
    ngrp = nblk // nbuf
    mesh = plsc.VectorSubcoreMesh(core_axis_name="c", subcore_axis_name="s")

    @functools.partial(
        pl.kernel,
        out_type=jax.ShapeDtypeStruct((n_rows, D), jnp.float32),
        mesh=mesh,
        scratch_types=[
            pltpu.VMEM((nblk, blk), jnp.int32),
            pltpu.VMEM((nbuf, blk, D), jnp.float32),
            pltpu.SemaphoreType.DMA,
            [pltpu.SemaphoreType.DMA] * nbuf,
        ],
    )
    def sc_gather(t_hbm, idx_hbm, out_hbm, idx_v, rows_v, gsem, wsems):
        wid = lax.axis_index("s") * nc + lax.axis_index("c")
        base = wid * rw
        pltpu.sync_copy(idx_hbm.at[wid], idx_v)

        for m in range(lookahead):
            pltpu.async_copy(t_hbm.at[idx_v.at[m]], rows_v.at[m % nbuf], gsem)

        def group(g, carry):
            for b in range(nbuf):
                j = g * nbuf + b
                m_b = (b + lookahead) % nbuf
                mj = j + lookahead

                @pl.when(mj < nblk)
                def _():
                    @pl.when(mj >= nbuf)
                    def _():
                        pltpu.make_async_copy(
                            rows_v.at[m_b],
                            out_hbm.at[pl.ds(base, blk)],
                            wsems[m_b],
                        ).wait()
                    pltpu.async_copy(t_hbm.at[idx_v.at[mj]],
                                     rows_v.at[m_b], gsem)

                pltpu.make_async_copy(
                    t_hbm.at[idx_v.at[j]], rows_v.at[b], gsem
                ).wait()
                pltpu.async_copy(rows_v.at[b],
                                 out_hbm.at[pl.ds(base + j * blk, blk)],
                                 wsems[b])
            return carry

        lax.fori_loop(0, ngrp, group, 0)

        for b in range(nbuf):
            pltpu.make_async_copy(
                rows_v.at[b], out_hbm.at[pl.ds(base, blk)], wsems[b]
            ).wait()

    return sc_gather, nw, nblk, blk


def kernel(card_indices, stages, visibility, order,
           rank_emb, suit_emb, stage_emb, visibility_emb, order_emb):
    B, L = card_indices.shape
    n = B * L
    rows = n // D

    card = card_indices.astype(jnp.int32).T.reshape(rows, D)
    st = stages.astype(jnp.int32).T.reshape(rows, D)
    vi = visibility.astype(jnp.int32).T.reshape(rows, D)
    od = order.astype(jnp.int32).T.reshape(rows, D)

    def pad8(t, r):
        return jnp.pad(t, ((0, r - t.shape[0]), (0, 0)))

    cat = jnp.concatenate([
        pad8(rank_emb, 16), pad8(suit_emb, 8), pad8(stage_emb, 8),
        pad8(visibility_emb, 8), pad8(order_emb, 8),
    ], axis=0)

    cidx, table = _prep(card, st, vi, od, cat)

    sc_gather, nw, nblk, blk = _make_sc_gather(n)
    idx3 = cidx.reshape(nw, nblk, blk)
    out = sc_gather(table, idx3)
    return out.reshape(L, B, D).transpose(1, 0, 2)


# speedup vs baseline: 38.8250x; 1.0025x over previous
import functools

import jax
import jax.numpy as jnp
from jax import lax
from jax.experimental import pallas as pl
from jax.experimental.pallas import tpu as pltpu
from jax.experimental.pallas import tpu_sc as plsc

D = 128
N_T = 3120
OFF_RANK, OFF_SUIT, OFF_STAGE, OFF_VIS, OFF_ORDER = 0, 16, 24, 32, 40
CAT_ROWS = 48


def _prep_body(card_ref, st_ref, vi_ref, od_ref, cat_ref, cidx_ref, t_ref):
    i = pl.program_id(0)
    cidx_ref[...] = (
        card_ref[...] * 60 + st_ref[...] * 15 + vi_ref[...] * 5 + od_ref[...]
    )
    @pl.when(i == 0)
    def _():
        row = lax.broadcasted_iota(jnp.int32, (N_T, 1), 0)
        card = row // 60
        svo = row - card * 60
        stage = svo // 15
        rem = svo - stage * 15
        vis = rem // 5
        order = rem - vis * 5
        cols = lax.broadcasted_iota(jnp.int32, (N_T, CAT_ROWS), 1)
        onehot = (
            (cols == OFF_RANK + card % 13).astype(jnp.float32)
            + (cols == OFF_SUIT + card // 13).astype(jnp.float32)
            + (cols == OFF_STAGE + stage).astype(jnp.float32)
            + (cols == OFF_VIS + vis).astype(jnp.float32)
            + (cols == OFF_ORDER + order).astype(jnp.float32)
        )
        t_ref[...] = jnp.dot(onehot, cat_ref[...],
                             preferred_element_type=jnp.float32)


def _prep(card, st, vi, od, cat):
    rows = card.shape[0]
    grid = 16
    blk = rows // grid
    return pl.pallas_call(
        _prep_body,
        grid=(grid,),
        in_specs=[
            pl.BlockSpec((blk, D), lambda i: (i, 0)),
            pl.BlockSpec((blk, D), lambda i: (i, 0)),
            pl.BlockSpec((blk, D), lambda i: (i, 0)),
            pl.BlockSpec((blk, D), lambda i: (i, 0)),
            pl.BlockSpec((CAT_ROWS, D), lambda i: (0, 0)),
        ],
        out_specs=[
            pl.BlockSpec((blk, D), lambda i: (i, 0)),
            pl.BlockSpec((N_T, D), lambda i: (0, 0)),
        ],
        out_shape=[
            jax.ShapeDtypeStruct((rows, D), jnp.int32),
            jax.ShapeDtypeStruct((N_T, D), jnp.float32),
        ],
    )(card, st, vi, od, cat)


def _make_sc_gather(n_rows):
    info = plsc.get_sparse_core_info()
    nc, ns = info.num_cores, info.num_subcores
    nw = nc * ns
    rw = n_rows // nw
    blk = 128
    nblk = rw // blk
    nbuf = 5
    lookahead = 4
    ngrp = nblk // nbuf
    mesh = plsc.VectorSubcoreMesh(core_axis_name="c", subcore_axis_name="s")

    @functools.partial(
        pl.kernel,
        out_type=jax.ShapeDtypeStruct((n_rows, D), jnp.float32),
        mesh=mesh,
        scratch_types=[
            pltpu.VMEM((nblk, blk), jnp.int32),
            pltpu.VMEM((nbuf, blk, D), jnp.float32),
            [pltpu.SemaphoreType.DMA] * nbuf,
            [pltpu.SemaphoreType.DMA] * nbuf,
        ],
    )
    def sc_gather(t_hbm, idx_hbm, out_hbm, idx_v, rows_v, gsems, wsems):
        wid = lax.axis_index("s") * nc + lax.axis_index("c")
        base = wid * rw
        pltpu.sync_copy(idx_hbm.at[wid], idx_v)

        for m in range(lookahead):
            pltpu.async_copy(t_hbm.at[idx_v.at[m]], rows_v.at[m % nbuf],
                             gsems[m % nbuf])

        def group(g, carry):
            for b in range(nbuf):
                j = g * nbuf + b
                m_b = (b + lookahead) % nbuf
                mj = j + lookahead

                @pl.when(mj < nblk)
                def _():
                    @pl.when(mj >= nbuf)
                    def _():
                        pltpu.make_async_copy(
                            rows_v.at[m_b],
                            out_hbm.at[pl.ds(base, blk)],
                            wsems[m_b],
                        ).wait()
                    pltpu.async_copy(t_hbm.at[idx_v.at[mj]],
                                     rows_v.at[m_b], gsems[m_b])

                pltpu.make_async_copy(
                    t_hbm.at[idx_v.at[j]], rows_v.at[b], gsems[b]
                ).wait()
                pltpu.async_copy(rows_v.at[b],
                                 out_hbm.at[pl.ds(base + j * blk, blk)],
                                 wsems[b])
            return carry

        lax.fori_loop(0, ngrp, group, 0)

        for b in range(nbuf):
            pltpu.make_async_copy(
                rows_v.at[b], out_hbm.at[pl.ds(base, blk)], wsems[b]
            ).wait()

    return sc_gather, nw, nblk, blk


def kernel(card_indices, stages, visibility, order,
           rank_emb, suit_emb, stage_emb, visibility_emb, order_emb):
    B, L = card_indices.shape
    n = B * L
    rows = n // D

    card = card_indices.astype(jnp.int32).T.reshape(rows, D)
    st = stages.astype(jnp.int32).T.reshape(rows, D)
    vi = visibility.astype(jnp.int32).T.reshape(rows, D)
    od = order.astype(jnp.int32).T.reshape(rows, D)

    def pad8(t, r):
        return jnp.pad(t, ((0, r - t.shape[0]), (0, 0)))

    cat = jnp.concatenate([
        pad8(rank_emb, 16), pad8(suit_emb, 8), pad8(stage_emb, 8),
        pad8(visibility_emb, 8), pad8(order_emb, 8),
    ], axis=0)

    cidx, table = _prep(card, st, vi, od, cat)

    sc_gather, nw, nblk, blk = _make_sc_gather(n)
    idx3 = cidx.reshape(nw, nblk, blk)
    out = sc_gather(table, idx3)
    return out.reshape(L, B, D).transpose(1, 0, 2)


# speedup vs baseline: 38.8323x; 1.0002x over previous
import functools

import jax
import jax.numpy as jnp
from jax import lax
from jax.experimental import pallas as pl
from jax.experimental.pallas import tpu as pltpu
from jax.experimental.pallas import tpu_sc as plsc

D = 128
N_T = 3120
OFF_RANK, OFF_SUIT, OFF_STAGE, OFF_VIS, OFF_ORDER = 0, 16, 24, 32, 40
CAT_ROWS = 48


def _prep_body(card_ref, st_ref, vi_ref, od_ref, cat_ref, cidx_ref, t_ref):
    i = pl.program_id(0)
    cidx_ref[...] = (
        card_ref[...] * 60 + st_ref[...] * 15 + vi_ref[...] * 5 + od_ref[...]
    )
    @pl.when(i == 0)
    def _():
        row = lax.broadcasted_iota(jnp.int32, (N_T, 1), 0)
        card = row // 60
        svo = row - card * 60
        stage = svo // 15
        rem = svo - stage * 15
        vis = rem // 5
        order = rem - vis * 5
        cols = lax.broadcasted_iota(jnp.int32, (N_T, CAT_ROWS), 1)
        onehot = (
            (cols == OFF_RANK + card % 13).astype(jnp.float32)
            + (cols == OFF_SUIT + card // 13).astype(jnp.float32)
            + (cols == OFF_STAGE + stage).astype(jnp.float32)
            + (cols == OFF_VIS + vis).astype(jnp.float32)
            + (cols == OFF_ORDER + order).astype(jnp.float32)
        )
        t_ref[...] = jnp.dot(onehot, cat_ref[...],
                             preferred_element_type=jnp.float32)


def _prep(card, st, vi, od, cat):
    rows = card.shape[0]
    grid = 16
    blk = rows // grid
    return pl.pallas_call(
        _prep_body,
        grid=(grid,),
        in_specs=[
            pl.BlockSpec((blk, D), lambda i: (i, 0)),
            pl.BlockSpec((blk, D), lambda i: (i, 0)),
            pl.BlockSpec((blk, D), lambda i: (i, 0)),
            pl.BlockSpec((blk, D), lambda i: (i, 0)),
            pl.BlockSpec((CAT_ROWS, D), lambda i: (0, 0)),
        ],
        out_specs=[
            pl.BlockSpec((blk, D), lambda i: (i, 0)),
            pl.BlockSpec((N_T, D), lambda i: (0, 0)),
        ],
        out_shape=[
            jax.ShapeDtypeStruct((rows, D), jnp.int32),
            jax.ShapeDtypeStruct((N_T, D), jnp.float32),
        ],
    )(card, st, vi, od, cat)


def _make_sc_gather(n_rows):
    info = plsc.get_sparse_core_info()
    nc, ns = info.num_cores, info.num_subcores
    nw = nc * ns
    rw = n_rows // nw
    blk = 128
    nblk = rw // blk
    nbuf = 5
    lookahead = 3
    ngrp = nblk // nbuf
    mesh = plsc.VectorSubcoreMesh(core_axis_name="c", subcore_axis_name="s")

    @functools.partial(
        pl.kernel,
        out_type=jax.ShapeDtypeStruct((n_rows, D), jnp.float32),
        mesh=mesh,
        scratch_types=[
            pltpu.VMEM((nblk, blk), jnp.int32),
            pltpu.VMEM((nbuf, blk, D), jnp.float32),
            [pltpu.SemaphoreType.DMA] * nbuf,
            [pltpu.SemaphoreType.DMA] * nbuf,
        ],
    )
    def sc_gather(t_hbm, idx_hbm, out_hbm, idx_v, rows_v, gsems, wsems):
        wid = lax.axis_index("s") * nc + lax.axis_index("c")
        base = wid * rw
        pltpu.sync_copy(idx_hbm.at[wid], idx_v)

        for m in range(lookahead):
            pltpu.async_copy(t_hbm.at[idx_v.at[m]], rows_v.at[m % nbuf],
                             gsems[m % nbuf])

        def group(g, carry):
            for b in range(nbuf):
                j = g * nbuf + b
                m_b = (b + lookahead) % nbuf
                mj = j + lookahead

                @pl.when(mj < nblk)
                def _():
                    @pl.when(mj >= nbuf)
                    def _():
                        pltpu.make_async_copy(
                            rows_v.at[m_b],
                            out_hbm.at[pl.ds(base, blk)],
                            wsems[m_b],
                        ).wait()
                    pltpu.async_copy(t_hbm.at[idx_v.at[mj]],
                                     rows_v.at[m_b], gsems[m_b])

                pltpu.make_async_copy(
                    t_hbm.at[idx_v.at[j]], rows_v.at[b], gsems[b]
                ).wait()
                pltpu.async_copy(rows_v.at[b],
                                 out_hbm.at[pl.ds(base + j * blk, blk)],
                                 wsems[b])
            return carry

        lax.fori_loop(0, ngrp, group, 0)

        for b in range(nbuf):
            pltpu.make_async_copy(
                rows_v.at[b], out_hbm.at[pl.ds(base, blk)], wsems[b]
            ).wait()

    return sc_gather, nw, nblk, blk


def kernel(card_indices, stages, visibility, order,
           rank_emb, suit_emb, stage_emb, visibility_emb, order_emb):
    B, L = card_indices.shape
    n = B * L
    rows = n // D

    card = card_indices.astype(jnp.int32).T.reshape(rows, D)
    st = stages.astype(jnp.int32).T.reshape(rows, D)
    vi = visibility.astype(jnp.int32).T.reshape(rows, D)
    od = order.astype(jnp.int32).T.reshape(rows, D)

    def pad8(t, r):
        return jnp.pad(t, ((0, r - t.shape[0]), (0, 0)))

    cat = jnp.concatenate([
        pad8(rank_emb, 16), pad8(suit_emb, 8), pad8(stage_emb, 8),
        pad8(visibility_emb, 8), pad8(order_emb, 8),
    ], axis=0)

    cidx, table = _prep(card, st, vi, od, cat)

    sc_gather, nw, nblk, blk = _make_sc_gather(n)
    idx3 = cidx.reshape(nw, nblk, blk)
    out = sc_gather(table, idx3)
    return out.reshape(L, B, D).transpose(1, 0, 2)


# speedup vs baseline: 78.7661x; 2.0284x over previous
import functools

import jax
import jax.numpy as jnp
from jax import lax
from jax.experimental import pallas as pl
from jax.experimental.pallas import tpu as pltpu
from jax.experimental.pallas import tpu_sc as plsc

D = 128
N_T = 3120
OFF_RANK, OFF_SUIT, OFF_STAGE, OFF_VIS, OFF_ORDER = 0, 16, 24, 32, 40
CAT_ROWS = 48


def _prep_body(card_ref, st_ref, vi_ref, od_ref, cat_ref, cidx_ref, t_ref):
    i = pl.program_id(0)
    cidx_ref[...] = (
        card_ref[...] * 60 + st_ref[...] * 15 + vi_ref[...] * 5 + od_ref[...]
    )
    @pl.when(i == 0)
    def _():
        row = lax.broadcasted_iota(jnp.int32, (N_T, 1), 0)
        card = row // 60
        svo = row - card * 60
        stage = svo // 15
        rem = svo - stage * 15
        vis = rem // 5
        order = rem - vis * 5
        cols = lax.broadcasted_iota(jnp.int32, (N_T, CAT_ROWS), 1)
        onehot = (
            (cols == OFF_RANK + card % 13).astype(jnp.float32)
            + (cols == OFF_SUIT + card // 13).astype(jnp.float32)
            + (cols == OFF_STAGE + stage).astype(jnp.float32)
            + (cols == OFF_VIS + vis).astype(jnp.float32)
            + (cols == OFF_ORDER + order).astype(jnp.float32)
        )
        t_ref[...] = jnp.dot(onehot, cat_ref[...],
                             preferred_element_type=jnp.float32)


def _prep(card, st, vi, od, cat):
    rows = card.shape[0]
    grid = 16
    blk = rows // grid
    return pl.pallas_call(
        _prep_body,
        grid=(grid,),
        in_specs=[
            pl.BlockSpec((blk, D), lambda i: (i, 0)),
            pl.BlockSpec((blk, D), lambda i: (i, 0)),
            pl.BlockSpec((blk, D), lambda i: (i, 0)),
            pl.BlockSpec((blk, D), lambda i: (i, 0)),
            pl.BlockSpec((CAT_ROWS, D), lambda i: (0, 0)),
        ],
        out_specs=[
            pl.BlockSpec((blk, D), lambda i: (i, 0)),
            pl.BlockSpec((N_T, D), lambda i: (0, 0)),
        ],
        out_shape=[
            jax.ShapeDtypeStruct((rows, D), jnp.int32),
            jax.ShapeDtypeStruct((N_T, D), jnp.float32),
        ],
    )(card, st, vi, od, cat)


def _make_sc_gather(n_rows):
    info = plsc.get_sparse_core_info()
    nc, ns = info.num_cores, info.num_subcores
    nw = nc * ns
    rw = n_rows // nw
    blk = 128
    nblk = rw // blk
    nbuf = 4
    lookahead = 3
    ngrp = nblk // nbuf
    mesh = plsc.VectorSubcoreMesh(core_axis_name="c", subcore_axis_name="s")

    @functools.partial(
        pl.kernel,
        out_type=jax.ShapeDtypeStruct((n_rows, D), jnp.float32),
        mesh=mesh,
        scratch_types=[
            pltpu.VMEM((nblk, blk), jnp.int32),
            pltpu.VMEM((nbuf, blk, D), jnp.float32),
            pltpu.VMEM_SHARED((N_T, D), jnp.float32),
            [pltpu.SemaphoreType.DMA] * nbuf,
            [pltpu.SemaphoreType.DMA] * nbuf,
        ],
    )
    def sc_gather(t_hbm, idx_hbm, out_hbm, idx_v, rows_v, spt, gsems, wsems):
        wid = lax.axis_index("s") * nc + lax.axis_index("c")
        base = wid * rw
        sid = lax.axis_index("s")

        toff = sid * 192
        for coff, csz in ((0, 128), (128, 64)):
            pltpu.sync_copy(t_hbm.at[pl.ds(toff + coff, csz)],
                            rows_v.at[0, pl.ds(0, csz)])
            pltpu.sync_copy(rows_v.at[0, pl.ds(0, csz)],
                            spt.at[pl.ds(toff + coff, csz)])

        @pl.when(sid == 0)
        def _():
            pltpu.sync_copy(t_hbm.at[pl.ds(16 * 192, 48)],
                            rows_v.at[0, pl.ds(0, 48)])
            pltpu.sync_copy(rows_v.at[0, pl.ds(0, 48)],
                            spt.at[pl.ds(16 * 192, 48)])
        pltpu.sync_copy(idx_hbm.at[wid], idx_v)
        plsc.subcore_barrier()

        for m in range(lookahead):
            pltpu.async_copy(spt.at[idx_v.at[m]], rows_v.at[m % nbuf],
                             gsems[m % nbuf])

        def group(g, carry):
            for b in range(nbuf):
                j = g * nbuf + b
                m_b = (b + lookahead) % nbuf
                mj = j + lookahead

                @pl.when(mj < nblk)
                def _():
                    @pl.when(mj >= nbuf)
                    def _():
                        pltpu.make_async_copy(
                            rows_v.at[m_b],
                            out_hbm.at[pl.ds(base, blk)],
                            wsems[m_b],
                        ).wait()
                    pltpu.async_copy(spt.at[idx_v.at[mj]],
                                     rows_v.at[m_b], gsems[m_b])

                pltpu.make_async_copy(
                    spt.at[idx_v.at[j]], rows_v.at[b], gsems[b]
                ).wait()
                pltpu.async_copy(rows_v.at[b],
                                 out_hbm.at[pl.ds(base + j * blk, blk)],
                                 wsems[b])
            return carry

        lax.fori_loop(0, ngrp, group, 0)

        for b in range(nbuf):
            pltpu.make_async_copy(
                rows_v.at[b], out_hbm.at[pl.ds(base, blk)], wsems[b]
            ).wait()

    return sc_gather, nw, nblk, blk


def kernel(card_indices, stages, visibility, order,
           rank_emb, suit_emb, stage_emb, visibility_emb, order_emb):
    B, L = card_indices.shape
    n = B * L
    rows = n // D

    card = card_indices.astype(jnp.int32).T.reshape(rows, D)
    st = stages.astype(jnp.int32).T.reshape(rows, D)
    vi = visibility.astype(jnp.int32).T.reshape(rows, D)
    od = order.astype(jnp.int32).T.reshape(rows, D)

    def pad8(t, r):
        return jnp.pad(t, ((0, r - t.shape[0]), (0, 0)))

    cat = jnp.concatenate([
        pad8(rank_emb, 16), pad8(suit_emb, 8), pad8(stage_emb, 8),
        pad8(visibility_emb, 8), pad8(order_emb, 8),
    ], axis=0)

    cidx, table = _prep(card, st, vi, od, cat)

    sc_gather, nw, nblk, blk = _make_sc_gather(n)
    idx3 = cidx.reshape(nw, nblk, blk)
    out = sc_gather(table, idx3)
    return out.reshape(L, B, D).transpose(1, 0, 2)


# speedup vs baseline: 84.2136x; 1.0692x over previous
import functools

import jax
import jax.numpy as jnp
from jax import lax
from jax.experimental import pallas as pl
from jax.experimental.pallas import tpu as pltpu
from jax.experimental.pallas import tpu_sc as plsc

D = 128
N_T = 3120
OFF_RANK, OFF_SUIT, OFF_STAGE, OFF_VIS, OFF_ORDER = 0, 16, 24, 32, 40
CAT_ROWS = 48


def _prep_body(card_ref, st_ref, vi_ref, od_ref, cat_ref, cidx_ref, t_ref):
    i = pl.program_id(0)
    L = card_ref.shape[0]
    bb = card_ref.shape[1] // D
    cidx = (
        card_ref[...] * 60 + st_ref[...] * 15 + vi_ref[...] * 5 + od_ref[...]
    )
    cidx_ref[...] = cidx.reshape(L, bb, D)
    @pl.when(i == 0)
    def _():
        row = lax.broadcasted_iota(jnp.int32, (N_T, 1), 0)
        card = row // 60
        svo = row - card * 60
        stage = svo // 15
        rem = svo - stage * 15
        vis = rem // 5
        order = rem - vis * 5
        cols = lax.broadcasted_iota(jnp.int32, (N_T, CAT_ROWS), 1)
        onehot = (
            (cols == OFF_RANK + card % 13).astype(jnp.float32)
            + (cols == OFF_SUIT + card // 13).astype(jnp.float32)
            + (cols == OFF_STAGE + stage).astype(jnp.float32)
            + (cols == OFF_VIS + vis).astype(jnp.float32)
            + (cols == OFF_ORDER + order).astype(jnp.float32)
        )
        t_ref[...] = jnp.dot(onehot, cat_ref[...],
                             preferred_element_type=jnp.float32)


def _prep(card, st, vi, od, cat):
    L, B = card.shape
    grid = 16
    bb = B // (grid * D)
    blk = bb * D
    ispec = pl.BlockSpec((L, blk), lambda i: (0, i))
    return pl.pallas_call(
        _prep_body,
        grid=(grid,),
        in_specs=[
            ispec, ispec, ispec, ispec,
            pl.BlockSpec((CAT_ROWS, D), lambda i: (0, 0)),
        ],
        out_specs=[
            pl.BlockSpec((L, bb, D), lambda i: (0, i, 0)),
            pl.BlockSpec((N_T, D), lambda i: (0, 0)),
        ],
        out_shape=[
            jax.ShapeDtypeStruct((L, B // D, D), jnp.int32),
            jax.ShapeDtypeStruct((N_T, D), jnp.float32),
        ],
    )(card, st, vi, od, cat)


def _make_sc_gather(n_rows):
    info = plsc.get_sparse_core_info()
    nc, ns = info.num_cores, info.num_subcores
    nw = nc * ns
    rw = n_rows // nw
    blk = 128
    nblk = rw // blk
    nbuf = 4
    lookahead = 3
    ngrp = nblk // nbuf
    mesh = plsc.VectorSubcoreMesh(core_axis_name="c", subcore_axis_name="s")

    @functools.partial(
        pl.kernel,
        out_type=jax.ShapeDtypeStruct((n_rows, D), jnp.float32),
        mesh=mesh,
        scratch_types=[
            pltpu.VMEM((nblk, blk), jnp.int32),
            pltpu.VMEM((nbuf, blk, D), jnp.float32),
            pltpu.VMEM_SHARED((N_T, D), jnp.float32),
            [pltpu.SemaphoreType.DMA] * nbuf,
            [pltpu.SemaphoreType.DMA] * nbuf,
        ],
    )
    def sc_gather(t_hbm, idx_hbm, out_hbm, idx_v, rows_v, spt, gsems, wsems):
        wid = lax.axis_index("s") * nc + lax.axis_index("c")
        base = wid * rw
        sid = lax.axis_index("s")

        toff = sid * 192
        for coff, csz in ((0, 128), (128, 64)):
            pltpu.sync_copy(t_hbm.at[pl.ds(toff + coff, csz)],
                            rows_v.at[0, pl.ds(0, csz)])
            pltpu.sync_copy(rows_v.at[0, pl.ds(0, csz)],
                            spt.at[pl.ds(toff + coff, csz)])

        @pl.when(sid == 0)
        def _():
            pltpu.sync_copy(t_hbm.at[pl.ds(16 * 192, 48)],
                            rows_v.at[0, pl.ds(0, 48)])
            pltpu.sync_copy(rows_v.at[0, pl.ds(0, 48)],
                            spt.at[pl.ds(16 * 192, 48)])
        pltpu.sync_copy(idx_hbm.at[wid], idx_v)
        plsc.subcore_barrier()

        for m in range(lookahead):
            pltpu.async_copy(spt.at[idx_v.at[m]], rows_v.at[m % nbuf],
                             gsems[m % nbuf])

        def group(g, carry):
            for b in range(nbuf):
                j = g * nbuf + b
                m_b = (b + lookahead) % nbuf
                mj = j + lookahead

                @pl.when(mj < nblk)
                def _():
                    @pl.when(mj >= nbuf)
                    def _():
                        pltpu.make_async_copy(
                            rows_v.at[m_b],
                            out_hbm.at[pl.ds(base, blk)],
                            wsems[m_b],
                        ).wait()
                    pltpu.async_copy(spt.at[idx_v.at[mj]],
                                     rows_v.at[m_b], gsems[m_b])

                pltpu.make_async_copy(
                    spt.at[idx_v.at[j]], rows_v.at[b], gsems[b]
                ).wait()
                pltpu.async_copy(rows_v.at[b],
                                 out_hbm.at[pl.ds(base + j * blk, blk)],
                                 wsems[b])
            return carry

        lax.fori_loop(0, ngrp, group, 0)

        for b in range(nbuf):
            pltpu.make_async_copy(
                rows_v.at[b], out_hbm.at[pl.ds(base, blk)], wsems[b]
            ).wait()

    return sc_gather, nw, nblk, blk


def kernel(card_indices, stages, visibility, order,
           rank_emb, suit_emb, stage_emb, visibility_emb, order_emb):
    B, L = card_indices.shape
    n = B * L

    card = card_indices.astype(jnp.int32).T
    st = stages.astype(jnp.int32).T
    vi = visibility.astype(jnp.int32).T
    od = order.astype(jnp.int32).T

    def pad8(t, r):
        return jnp.pad(t, ((0, r - t.shape[0]), (0, 0)))

    cat = jnp.concatenate([
        pad8(rank_emb, 16), pad8(suit_emb, 8), pad8(stage_emb, 8),
        pad8(visibility_emb, 8), pad8(order_emb, 8),
    ], axis=0)

    cidx, table = _prep(card, st, vi, od, cat)

    sc_gather, nw, nblk, blk = _make_sc_gather(n)
    idx3 = cidx.reshape(nw, nblk, blk)
    out = sc_gather(table, idx3)
    return out.reshape(L, B, D).transpose(1, 0, 2)
